# baseline (speedup 1.0000x reference)
import functools

import jax
import jax.numpy as jnp
from jax import lax
from jax.experimental import pallas as pl
from jax.experimental.pallas import tpu as pltpu
from jax.experimental.pallas import tpu_sc as plsc

B = 16384
EMB = 64
VEMB = 32
NVIS = 2048
LANES = 128
UPG = LANES // EMB
VPG = LANES // VEMB
NC = 2
NS = 16
NW = NC * NS
BPW = B // NW
CH = 128
NCHUNK = BPW // CH


def _sc_gather(up_idx, ipp_idx, inp_idx, uvp_idx, ut2, it2, uvt2):
    mesh = plsc.VectorSubcoreMesh(core_axis_name="c", subcore_axis_name="s")

    @functools.partial(
        pl.kernel,
        out_type=[
            jax.ShapeDtypeStruct((B, LANES), jnp.float32),
            jax.ShapeDtypeStruct((B, LANES), jnp.float32),
            jax.ShapeDtypeStruct((B, LANES), jnp.float32),
            jax.ShapeDtypeStruct((B, LANES), jnp.float32),
        ],
        mesh=mesh,
        compiler_params=pltpu.CompilerParams(use_tc_tiling_on_sc=True),
        scratch_types=[
            pltpu.VMEM((BPW,), jnp.int32),
            pltpu.VMEM((BPW,), jnp.int32),
            pltpu.VMEM((BPW,), jnp.int32),
            pltpu.VMEM((BPW,), jnp.int32),
            pltpu.VMEM((CH, LANES), jnp.float32),
            pltpu.VMEM((CH, LANES), jnp.float32),
            pltpu.VMEM((CH, LANES), jnp.float32),
            pltpu.VMEM((CH, LANES), jnp.float32),
            pltpu.SemaphoreType.DMA,
        ],
    )
    def body(up_hbm, ipp_hbm, inp_hbm, uvp_hbm, ut_hbm, it_hbm, uvt_hbm,
             uf_out, ipf_out, inf_out, uvf_out,
             up_v, ipp_v, inp_v, uvp_v, u_b, ip_b, in_b, uv_b, sem):
        wid = lax.axis_index("s") * NC + lax.axis_index("c")
        base = wid * BPW
        pltpu.sync_copy(up_hbm.at[pl.ds(base, BPW)], up_v)
        pltpu.sync_copy(ipp_hbm.at[pl.ds(base, BPW)], ipp_v)
        pltpu.sync_copy(inp_hbm.at[pl.ds(base, BPW)], inp_v)
        pltpu.sync_copy(uvp_hbm.at[pl.ds(base, BPW)], uvp_v)
        for c in range(NCHUNK):
            o = c * CH
            c1 = pltpu.async_copy(ut_hbm.at[up_v.at[pl.ds(o, CH)]], u_b, sem)
            c2 = pltpu.async_copy(it_hbm.at[ipp_v.at[pl.ds(o, CH)]], ip_b, sem)
            c3 = pltpu.async_copy(it_hbm.at[inp_v.at[pl.ds(o, CH)]], in_b, sem)
            c4 = pltpu.async_copy(uvt_hbm.at[uvp_v.at[pl.ds(o, CH)]], uv_b, sem)
            c1.wait()
            c2.wait()
            c3.wait()
            c4.wait()
            pltpu.sync_copy(u_b, uf_out.at[pl.ds(base + o, CH)])
            pltpu.sync_copy(ip_b, ipf_out.at[pl.ds(base + o, CH)])
            pltpu.sync_copy(in_b, inf_out.at[pl.ds(base + o, CH)])
            pltpu.sync_copy(uv_b, uvf_out.at[pl.ds(base + o, CH)])

    return body(up_idx, ipp_idx, inp_idx, uvp_idx, ut2, it2, uvt2)


_BLK = 1024


def _tc_matmul_body(vfp_ref, vfn_ref, w_ref, vp_ref, vn_ref):
    w = w_ref[...]
    dims = (((1,), (1,)), ((), ()))
    vp_ref[...] = lax.dot_general(vfp_ref[...], w, dims,
                                  preferred_element_type=jnp.float32)
    vn_ref[...] = lax.dot_general(vfn_ref[...], w, dims,
                                  preferred_element_type=jnp.float32)


def _tc_matmul(vfp, vfn, w):
    return pl.pallas_call(
        _tc_matmul_body,
        grid=(B // _BLK,),
        in_specs=[
            pl.BlockSpec((_BLK, NVIS), lambda i: (i, 0)),
            pl.BlockSpec((_BLK, NVIS), lambda i: (i, 0)),
            pl.BlockSpec((VEMB, NVIS), lambda i: (0, 0)),
        ],
        out_specs=[
            pl.BlockSpec((_BLK, VEMB), lambda i: (i, 0)),
            pl.BlockSpec((_BLK, VEMB), lambda i: (i, 0)),
        ],
        out_shape=[
            jax.ShapeDtypeStruct((B, VEMB), jnp.float32),
            jax.ShapeDtypeStruct((B, VEMB), jnp.float32),
        ],
    )(vfp, vfn, w)


def _tc_combine_body(uf_ref, ipf_ref, inf_ref, uvf_ref,
                     us_ref, ips_ref, ins_ref, uvs_ref,
                     vp_ref, vn_ref, pos_ref, neg_ref):
    uf = uf_ref[...]
    u = jnp.where(us_ref[...] == 0, uf[:, :EMB], uf[:, EMB:])
    ipf = ipf_ref[...]
    ipe = jnp.where(ips_ref[...] == 0, ipf[:, :EMB], ipf[:, EMB:])
    inf = inf_ref[...]
    ine = jnp.where(ins_ref[...] == 0, inf[:, :EMB], inf[:, EMB:])
    uvf = uvf_ref[...]
    uvs = uvs_ref[...]
    uv = jnp.where(uvs == 0, uvf[:, 0:VEMB],
                   jnp.where(uvs == 1, uvf[:, VEMB:2 * VEMB],
                             jnp.where(uvs == 2, uvf[:, 2 * VEMB:3 * VEMB],
                                       uvf[:, 3 * VEMB:])))
    gp = jnp.sum(u * ipe, axis=1, keepdims=True)
    gn = jnp.sum(u * ine, axis=1, keepdims=True)
    pos_ref[...] = gp + jnp.sum(uv * vp_ref[...], axis=1, keepdims=True)
    neg_ref[...] = gn + jnp.sum(uv * vn_ref[...], axis=1, keepdims=True)


def _tc_combine(uf, ipf, inf, uvf, us, ips, ins, uvs, vp, vn):
    wide = pl.BlockSpec((_BLK, LANES), lambda i: (i, 0))
    narrow = pl.BlockSpec((_BLK, VEMB), lambda i: (i, 0))
    sel = pl.BlockSpec((_BLK, 1), lambda i: (i, 0))
    one = pl.BlockSpec((_BLK, 1), lambda i: (i, 0))
    pos, neg = pl.pallas_call(
        _tc_combine_body,
        grid=(B // _BLK,),
        in_specs=[wide, wide, wide, wide, sel, sel, sel, sel, narrow, narrow],
        out_specs=[one, one],
        out_shape=[
            jax.ShapeDtypeStruct((B, 1), jnp.float32),
            jax.ShapeDtypeStruct((B, 1), jnp.float32),
        ],
    )(uf, ipf, inf, uvf, us.reshape(B, 1), ips.reshape(B, 1),
      ins.reshape(B, 1), uvs.reshape(B, 1), vp, vn)
    return pos[:, 0], neg[:, 0]


def kernel(user_indices, item_pos_indices, item_neg_indices,
           visual_features_pos, visual_features_neg,
           user_table, item_table, W_visual, user_visual_table):
    u_idx = user_indices.astype(jnp.int32)
    ip_idx = item_pos_indices.astype(jnp.int32)
    in_idx = item_neg_indices.astype(jnp.int32)
    nu = user_table.shape[0]
    ni = item_table.shape[0]
    ut2 = user_table.reshape(nu // UPG, LANES)
    it2 = item_table.reshape(ni // UPG, LANES)
    uvt2 = user_visual_table.reshape(nu // VPG, LANES)
    vp, vn = _tc_matmul(visual_features_pos, visual_features_neg, W_visual)
    uf, ipf, inf, uvf = _sc_gather(
        u_idx // UPG, ip_idx // UPG, in_idx // UPG, u_idx // VPG,
        ut2, it2, uvt2)
    return _tc_combine(uf, ipf, inf, uvf,
                       u_idx % UPG, ip_idx % UPG, in_idx % UPG, u_idx % VPG,
                       vp, vn)


# speedup vs baseline: 1.0309x; 1.0309x over previous
import functools

import jax
import jax.numpy as jnp
from jax import lax
from jax.experimental import pallas as pl
from jax.experimental.pallas import tpu as pltpu
from jax.experimental.pallas import tpu_sc as plsc

B = 16384
EMB = 64
VEMB = 32
NVIS = 2048
NC = 2
NS = 16
NW = NC * NS
BPW = B // NW
NGRP = BPW // 16


def _sc_gather_dots(u_idx, ip_idx, in_idx, user_table, item_table, uv_table):
    mesh = plsc.VectorSubcoreMesh(core_axis_name="c", subcore_axis_name="s")

    @functools.partial(
        pl.kernel,
        out_type=[
            jax.ShapeDtypeStruct((B,), jnp.float32),
            jax.ShapeDtypeStruct((B,), jnp.float32),
            jax.ShapeDtypeStruct((B, VEMB), jnp.float32),
        ],
        mesh=mesh,
        compiler_params=pltpu.CompilerParams(use_tc_tiling_on_sc=False,
                                             needs_layout_passes=False),
        scratch_types=[
            pltpu.VMEM((BPW,), jnp.int32),
            pltpu.VMEM((BPW,), jnp.int32),
            pltpu.VMEM((BPW,), jnp.int32),
            pltpu.VMEM((BPW, EMB), jnp.float32),
            pltpu.VMEM((BPW, EMB), jnp.float32),
            pltpu.VMEM((BPW, EMB), jnp.float32),
            pltpu.VMEM((BPW, VEMB), jnp.float32),
            pltpu.VMEM((BPW,), jnp.float32),
            pltpu.VMEM((BPW,), jnp.float32),
            pltpu.SemaphoreType.DMA,
        ],
    )
    def body(uidx_hbm, ipidx_hbm, inidx_hbm, ut_hbm, it_hbm, uvt_hbm,
             gp_out, gn_out, uv_out,
             uidx_v, ipidx_v, inidx_v, u_v, ip_v, in_v, uv_v, gp_v, gn_v,
             sem):
        wid = lax.axis_index("s") * NC + lax.axis_index("c")
        base = wid * BPW
        pltpu.sync_copy(uidx_hbm.at[pl.ds(base, BPW)], uidx_v)
        pltpu.sync_copy(ipidx_hbm.at[pl.ds(base, BPW)], ipidx_v)
        pltpu.sync_copy(inidx_hbm.at[pl.ds(base, BPW)], inidx_v)
        c1 = pltpu.async_copy(ut_hbm.at[uidx_v], u_v, sem)
        c2 = pltpu.async_copy(it_hbm.at[ipidx_v], ip_v, sem)
        c3 = pltpu.async_copy(it_hbm.at[inidx_v], in_v, sem)
        c4 = pltpu.async_copy(uvt_hbm.at[uidx_v], uv_v, sem)
        c1.wait()
        c2.wait()
        c3.wait()
        c4.wait()

        for g in range(NGRP):
            rows = g * 16 + lax.iota(jnp.int32, 16)

            def dot_body(d, carry):
                accp, accn = carry
                cols = jnp.full((16,), d, jnp.int32)
                u16 = plsc.load_gather(u_v, [rows, cols])
                accp = accp + u16 * plsc.load_gather(ip_v, [rows, cols])
                accn = accn + u16 * plsc.load_gather(in_v, [rows, cols])
                return accp, accn

            zeros = jnp.zeros((16,), jnp.float32)
            accp, accn = lax.fori_loop(0, EMB, dot_body, (zeros, zeros))
            gp_v[pl.ds(g * 16, 16)] = accp
            gn_v[pl.ds(g * 16, 16)] = accn

        pltpu.sync_copy(gp_v, gp_out.at[pl.ds(base, BPW)])
        pltpu.sync_copy(gn_v, gn_out.at[pl.ds(base, BPW)])
        pltpu.sync_copy(uv_v, uv_out.at[pl.ds(base, BPW)])

    return body(u_idx, ip_idx, in_idx, user_table, item_table, uv_table)


_BLK = 1024


def _tc_matmul_body(vfp_ref, vfn_ref, w_ref, vp_ref, vn_ref):
    w = w_ref[...]
    dims = (((1,), (1,)), ((), ()))
    vp_ref[...] = lax.dot_general(vfp_ref[...], w, dims,
                                  preferred_element_type=jnp.float32)
    vn_ref[...] = lax.dot_general(vfn_ref[...], w, dims,
                                  preferred_element_type=jnp.float32)


def _tc_matmul(vfp, vfn, w):
    return pl.pallas_call(
        _tc_matmul_body,
        grid=(B // _BLK,),
        in_specs=[
            pl.BlockSpec((_BLK, NVIS), lambda i: (i, 0)),
            pl.BlockSpec((_BLK, NVIS), lambda i: (i, 0)),
            pl.BlockSpec((VEMB, NVIS), lambda i: (0, 0)),
        ],
        out_specs=[
            pl.BlockSpec((_BLK, VEMB), lambda i: (i, 0)),
            pl.BlockSpec((_BLK, VEMB), lambda i: (i, 0)),
        ],
        out_shape=[
            jax.ShapeDtypeStruct((B, VEMB), jnp.float32),
            jax.ShapeDtypeStruct((B, VEMB), jnp.float32),
        ],
    )(vfp, vfn, w)


def _tc_combine_body(vp_ref, vn_ref, uv_ref, gp_ref, gn_ref,
                     pos_ref, neg_ref):
    uv = uv_ref[...]
    pos_ref[...] = gp_ref[...] + jnp.sum(uv * vp_ref[...], axis=1,
                                         keepdims=True)
    neg_ref[...] = gn_ref[...] + jnp.sum(uv * vn_ref[...], axis=1,
                                         keepdims=True)


def _tc_combine(vp, vn, uv, gp, gn):
    pos, neg = pl.pallas_call(
        _tc_combine_body,
        grid=(B // _BLK,),
        in_specs=[
            pl.BlockSpec((_BLK, VEMB), lambda i: (i, 0)),
            pl.BlockSpec((_BLK, VEMB), lambda i: (i, 0)),
            pl.BlockSpec((_BLK, VEMB), lambda i: (i, 0)),
            pl.BlockSpec((_BLK, 1), lambda i: (i, 0)),
            pl.BlockSpec((_BLK, 1), lambda i: (i, 0)),
        ],
        out_specs=[
            pl.BlockSpec((_BLK, 1), lambda i: (i, 0)),
            pl.BlockSpec((_BLK, 1), lambda i: (i, 0)),
        ],
        out_shape=[
            jax.ShapeDtypeStruct((B, 1), jnp.float32),
            jax.ShapeDtypeStruct((B, 1), jnp.float32),
        ],
    )(vp, vn, uv, gp.reshape(B, 1), gn.reshape(B, 1))
    return pos[:, 0], neg[:, 0]


def kernel(user_indices, item_pos_indices, item_neg_indices,
           visual_features_pos, visual_features_neg,
           user_table, item_table, W_visual, user_visual_table):
    u_idx = user_indices.astype(jnp.int32)
    ip_idx = item_pos_indices.astype(jnp.int32)
    in_idx = item_neg_indices.astype(jnp.int32)
    vp, vn = _tc_matmul(visual_features_pos, visual_features_neg, W_visual)
    gp, gn, uv = _sc_gather_dots(u_idx, ip_idx, in_idx,
                                 user_table, item_table, user_visual_table)
    return _tc_combine(vp, vn, uv, gp, gn)
